# Initial kernel scaffold; baseline (speedup 1.0000x reference)
import jax
import jax.numpy as jnp
from jax.experimental import pallas as pl


def kernel(x, W, b):
    raise NotImplementedError("write your pallas kernel here")


# speedup vs baseline: 4.7982x; 4.7982x over previous
import jax
import jax.numpy as jnp
from jax.experimental import pallas as pl

D_MODEL = 2048
NUM_EXPERTS = 64
N_TOKENS = 8192
CONFIDENCE_THRESHOLD = 0.5
TOKEN_TILE = 256


def _router_kernel(x_ref, w_ref, b_ref, rw_ref, probs_ref, cnt_ref):
    logits = jnp.dot(x_ref[...], w_ref[...],
                     preferred_element_type=jnp.float32,
                     precision=jax.lax.Precision.DEFAULT)
    logits = logits + b_ref[...]

    m = jnp.max(logits, axis=-1, keepdims=True)
    e = jnp.exp(logits - m)
    p = e / jnp.sum(e, axis=-1, keepdims=True)

    p_e = p[:, :, None]
    p_j = p[:, None, :]
    e_idx = jax.lax.broadcasted_iota(jnp.int32, (1, NUM_EXPERTS, NUM_EXPERTS), 1)
    j_idx = jax.lax.broadcasted_iota(jnp.int32, (1, NUM_EXPERTS, NUM_EXPERTS), 2)
    before = (p_j > p_e) | ((p_j == p_e) & (j_idx < e_idx))
    s = jnp.sum(jnp.where(before, p_j, 0.0), axis=2)

    active = s < CONFIDENCE_THRESHOLD
    active_probs = jnp.where(active, p, 0.0)
    mass = jnp.sum(active_probs, axis=-1, keepdims=True)
    rw_ref[...] = active_probs / (mass + 1e-6)
    probs_ref[...] = p
    cnt_ref[...] = jnp.sum(active.astype(jnp.int32), axis=-1, keepdims=True)


def kernel(x, W, b):
    n_tiles = N_TOKENS // TOKEN_TILE
    b2 = b.reshape(1, NUM_EXPERTS)
    rw, probs, cnt = pl.pallas_call(
        _router_kernel,
        grid=(n_tiles,),
        in_specs=[
            pl.BlockSpec((TOKEN_TILE, D_MODEL), lambda i: (i, 0)),
            pl.BlockSpec((D_MODEL, NUM_EXPERTS), lambda i: (0, 0)),
            pl.BlockSpec((1, NUM_EXPERTS), lambda i: (0, 0)),
        ],
        out_specs=[
            pl.BlockSpec((TOKEN_TILE, NUM_EXPERTS), lambda i: (i, 0)),
            pl.BlockSpec((TOKEN_TILE, NUM_EXPERTS), lambda i: (i, 0)),
            pl.BlockSpec((TOKEN_TILE, 1), lambda i: (i, 0)),
        ],
        out_shape=[
            jax.ShapeDtypeStruct((N_TOKENS, NUM_EXPERTS), jnp.float32),
            jax.ShapeDtypeStruct((N_TOKENS, NUM_EXPERTS), jnp.float32),
            jax.ShapeDtypeStruct((N_TOKENS, 1), jnp.int32),
        ],
    )(x, W, b2)
    return rw, probs, cnt.reshape(N_TOKENS)


# speedup vs baseline: 7.7099x; 1.6068x over previous
import jax
import jax.numpy as jnp
from jax.experimental import pallas as pl
from jax.experimental.pallas import tpu as pltpu

D_MODEL = 2048
NUM_EXPERTS = 64
N_TOKENS = 8192
CONFIDENCE_THRESHOLD = 0.5
TOKEN_TILE = 256


def _sort_desc_lanes(v):
    n = NUM_EXPERTS
    idx = jax.lax.broadcasted_iota(jnp.int32, (1, n), 1)
    k = 2
    while k <= n:
        d = (idx & k) != 0
        j = k // 2
        while j >= 1:
            m = (idx & j) != 0
            pv = jnp.where(m, pltpu.roll(v, j, 1), pltpu.roll(v, n - j, 1))
            v = jnp.where(m == d, jnp.maximum(v, pv), jnp.minimum(v, pv))
            j //= 2
        k *= 2
    return v


def _cumsum_lanes(v):
    n = NUM_EXPERTS
    idx = jax.lax.broadcasted_iota(jnp.int32, (1, n), 1)
    s = 1
    while s < n:
        v = v + jnp.where(idx >= s, pltpu.roll(v, s, 1), 0.0)
        s *= 2
    return v


def _router_kernel(x_ref, w_ref, b_ref, rw_ref, probs_ref, cnt_ref):
    logits = jnp.dot(x_ref[...], w_ref[...],
                     preferred_element_type=jnp.float32,
                     precision=jax.lax.Precision.DEFAULT)
    logits = logits + b_ref[...]

    mx = jnp.max(logits, axis=-1, keepdims=True)
    ex = jnp.exp(logits - mx)
    p = ex / jnp.sum(ex, axis=-1, keepdims=True)

    sp = _sort_desc_lanes(p)
    shifted = _cumsum_lanes(sp) - sp
    act_s = shifted < CONFIDENCE_THRESHOLD
    mass = jnp.sum(jnp.where(act_s, sp, 0.0), axis=-1, keepdims=True)
    theta = jnp.min(jnp.where(act_s, sp, jnp.inf), axis=-1, keepdims=True)
    r = jnp.sum((act_s & (sp == theta)).astype(jnp.float32),
                axis=-1, keepdims=True)

    eqf = (p == theta).astype(jnp.float32)
    rank_excl = _cumsum_lanes(eqf) - eqf
    active = (p > theta) | ((p == theta) & (rank_excl < r))

    active_probs = jnp.where(active, p, 0.0)
    rw_ref[...] = active_probs / (mass + 1e-6)
    probs_ref[...] = p
    cnt_ref[...] = jnp.sum(active.astype(jnp.int32), axis=-1, keepdims=True)


def kernel(x, W, b):
    n_tiles = N_TOKENS // TOKEN_TILE
    b2 = b.reshape(1, NUM_EXPERTS)
    rw, probs, cnt = pl.pallas_call(
        _router_kernel,
        grid=(n_tiles,),
        in_specs=[
            pl.BlockSpec((TOKEN_TILE, D_MODEL), lambda i: (i, 0)),
            pl.BlockSpec((D_MODEL, NUM_EXPERTS), lambda i: (0, 0)),
            pl.BlockSpec((1, NUM_EXPERTS), lambda i: (0, 0)),
        ],
        out_specs=[
            pl.BlockSpec((TOKEN_TILE, NUM_EXPERTS), lambda i: (i, 0)),
            pl.BlockSpec((TOKEN_TILE, NUM_EXPERTS), lambda i: (i, 0)),
            pl.BlockSpec((TOKEN_TILE, 1), lambda i: (i, 0)),
        ],
        out_shape=[
            jax.ShapeDtypeStruct((N_TOKENS, NUM_EXPERTS), jnp.float32),
            jax.ShapeDtypeStruct((N_TOKENS, NUM_EXPERTS), jnp.float32),
            jax.ShapeDtypeStruct((N_TOKENS, 1), jnp.int32),
        ],
    )(x, W, b2)
    return rw, probs, cnt.reshape(N_TOKENS)


# speedup vs baseline: 11.7752x; 1.5273x over previous
import jax
import jax.numpy as jnp
from jax.experimental import pallas as pl
from jax.experimental.pallas import tpu as pltpu

D_MODEL = 2048
NUM_EXPERTS = 64
N_TOKENS = 8192
CONFIDENCE_THRESHOLD = 0.5
TOKEN_TILE = 1024
PAIR_ROWS = TOKEN_TILE // 2
LANES = 2 * NUM_EXPERTS


def _lane_group_iota():
    return jax.lax.broadcasted_iota(jnp.int32, (1, LANES), 1) & (NUM_EXPERTS - 1)


def _sort_desc_groups(v):
    idx = _lane_group_iota()
    k = 2
    while k <= NUM_EXPERTS:
        d = (idx & k) != 0
        j = k // 2
        while j >= 1:
            m = (idx & j) != 0
            pv = jnp.where(m, pltpu.roll(v, j, 1), pltpu.roll(v, LANES - j, 1))
            v = jnp.where(m == d, jnp.maximum(v, pv), jnp.minimum(v, pv))
            j //= 2
        k *= 2
    return v


def _cumsum_groups(v):
    idx = _lane_group_iota()
    s = 1
    while s < NUM_EXPERTS:
        v = v + jnp.where(idx >= s, pltpu.roll(v, s, 1), 0.0)
        s *= 2
    return v


def _butterfly(v, combine):
    idx = _lane_group_iota()
    s = 1
    while s < NUM_EXPERTS:
        pv = jnp.where((idx & s) != 0,
                       pltpu.roll(v, s, 1), pltpu.roll(v, LANES - s, 1))
        v = combine(v, pv)
        s *= 2
    return v


def _router_kernel(x_ref, w_ref, b_ref, rw_ref, probs_ref, cnt_ref):
    logits = jnp.dot(x_ref[...], w_ref[...],
                     preferred_element_type=jnp.float32,
                     precision=jax.lax.Precision.DEFAULT)
    logits = logits + b_ref[...]

    gi = jax.lax.broadcasted_iota(jnp.int32, (LANES, LANES), 0)
    gj = jax.lax.broadcasted_iota(jnp.int32, (LANES, LANES), 1)
    bd = ((gi // NUM_EXPERTS) == (gj // NUM_EXPERTS)).astype(jnp.float32)

    def gsum(a):
        return jnp.dot(a, bd, preferred_element_type=jnp.float32,
                       precision=jax.lax.Precision.HIGHEST)

    mx = _butterfly(logits, jnp.maximum)
    ex = jnp.exp(logits - mx)
    p = ex / gsum(ex)

    sp = _sort_desc_groups(p)
    shifted = _cumsum_groups(sp) - sp
    act_s = shifted < CONFIDENCE_THRESHOLD
    act_p_s = jnp.where(act_s, sp, 0.0)
    mass = gsum(act_p_s)
    theta = _butterfly(jnp.where(act_s, sp, jnp.inf), jnp.minimum)
    r = gsum((act_s & (sp == theta)).astype(jnp.float32))

    eqf = (p == theta).astype(jnp.float32)
    rank_excl = _cumsum_groups(eqf) - eqf
    active = (p > theta) | ((p == theta) & (rank_excl < r))

    active_probs = jnp.where(active, p, 0.0)
    rw_ref[...] = active_probs / (mass + 1e-6)
    probs_ref[...] = p
    cnt_ref[...] = gsum(active.astype(jnp.float32)).astype(jnp.int32)


def kernel(x, W, b):
    n_tiles = N_TOKENS // TOKEN_TILE
    xp = x.reshape(N_TOKENS // 2, 2 * D_MODEL)
    w2 = jnp.zeros((2 * D_MODEL, LANES), dtype=W.dtype)
    w2 = w2.at[:D_MODEL, :NUM_EXPERTS].set(W)
    w2 = w2.at[D_MODEL:, NUM_EXPERTS:].set(W)
    b2 = jnp.concatenate([b, b]).reshape(1, LANES)
    rw, probs, cnt = pl.pallas_call(
        _router_kernel,
        grid=(n_tiles,),
        in_specs=[
            pl.BlockSpec((PAIR_ROWS, 2 * D_MODEL), lambda i: (i, 0)),
            pl.BlockSpec((2 * D_MODEL, LANES), lambda i: (0, 0)),
            pl.BlockSpec((1, LANES), lambda i: (0, 0)),
        ],
        out_specs=[
            pl.BlockSpec((PAIR_ROWS, LANES), lambda i: (i, 0)),
            pl.BlockSpec((PAIR_ROWS, LANES), lambda i: (i, 0)),
            pl.BlockSpec((PAIR_ROWS, LANES), lambda i: (i, 0)),
        ],
        out_shape=[
            jax.ShapeDtypeStruct((N_TOKENS // 2, LANES), jnp.float32),
            jax.ShapeDtypeStruct((N_TOKENS // 2, LANES), jnp.float32),
            jax.ShapeDtypeStruct((N_TOKENS // 2, LANES), jnp.int32),
        ],
    )(xp, w2, b2)
    rw = rw.reshape(N_TOKENS, NUM_EXPERTS)
    probs = probs.reshape(N_TOKENS, NUM_EXPERTS)
    cnt = cnt.reshape(N_TOKENS, NUM_EXPERTS)[:, 0]
    return rw, probs, cnt
